# baseline (speedup 1.0000x reference)
import jax
import jax.numpy as jnp
from jax import lax
from jax.experimental import pallas as pl
from jax.experimental.pallas import tpu as pltpu
from jax.experimental.pallas import tpu_sc as plsc

_MARGIN = 0.3
_K = 8
_N = 512
_L = 16
_NC = 2
_NS = 16
_NW = _NC * _NS
_RPW = _N // _NW
_NCHUNK = _N // _L


def _sc_body(tbl_ref, tgt_ref, out_ref, t_v, rows_v, sbuf_v, loss_v, sem):
    cid = lax.axis_index("c")
    sid = lax.axis_index("s")
    wid = sid * _NC + cid
    base = wid * _RPW

    pltpu.sync_copy(tgt_ref, t_v)
    idxv = t_v[pl.ds(base, _L)]
    pltpu.async_copy(tbl_ref.at[idxv], rows_v, sem).wait()

    iota = lax.iota(jnp.int32, _L)
    lane_lt8 = iota < _K

    pos_inf = jnp.full((_L,), jnp.inf, jnp.float32)
    neg_inf = jnp.full((_L,), -jnp.inf, jnp.float32)
    inf_s = jnp.float32(jnp.inf)

    def _merge(best, ch, thr):
        del thr
        out = jnp.sort(jnp.minimum(best, lax.rev(jnp.sort(ch), (0,))))
        return out, jnp.max(out)

    def _keep(best, ch, thr):
        del ch
        return best, thr

    def row_body(l, acc):
        lane_l = iota == l
        ti = jnp.sum(jnp.where(lane_l, idxv, 0))

        def chunk(c, carry):
            mx, mn, cnt, b2, thr2 = carry
            v = rows_v[l, pl.ds(c * _L, _L)]
            tc_ = t_v[pl.ds(c * _L, _L)]
            m = tc_ == ti
            mx = jnp.maximum(mx, v)
            mn = jnp.minimum(mn, v)
            plsc.store_compressed(sbuf_v.at[pl.ds(cnt, _L)], v, mask=m)
            cnt = cnt + plsc.all_reduce_population_count(m)[0]
            m2n = jnp.where(m, jnp.inf, -v)
            c2 = plsc.all_reduce_population_count(m2n < thr2)[0] > 0
            b2, thr2 = lax.cond(c2, _merge, _keep, b2, m2n, thr2)
            return mx, mn, cnt, b2, thr2

        mx, mn, cnt, b2, _ = lax.fori_loop(
            0, _NCHUNK, chunk,
            (neg_inf, pos_inf, jnp.int32(0), pos_inf, inf_s))
        max1 = jnp.max(mx)
        min2 = jnp.min(mn)

        sbuf_v[pl.ds(cnt, _L)] = pos_inf

        def merge1(c, b1):
            ch = sbuf_v[pl.ds(c * _L, _L)]
            return jnp.sort(jnp.minimum(b1, lax.rev(jnp.sort(ch), (0,))))

        nm = (cnt + (_L - 1)) // _L
        b1 = lax.fori_loop(0, nm, merge1, pos_inf)
        tmp1 = jnp.minimum(b1, max1)
        tmp2 = jnp.maximum(-b2, min2)

        g_vec = rows_v[l, pl.ds(base, _L)]
        g = jnp.sum(jnp.where(lane_l, g_vec, 0.0))
        ap = jnp.abs(g - tmp1)
        an = g - tmp2

        for bi in range(_K):
            t = jnp.maximum(ap - an[bi] + _MARGIN, 0.0)
            acc = acc + jnp.where(lane_lt8, t, 0.0)
        return acc

    acc = lax.fori_loop(0, _RPW, row_body, jnp.zeros((_L,), jnp.float32))
    loss_v[...] = acc * (1.0 / (_N * _K * _K))
    pltpu.sync_copy(loss_v, out_ref.at[wid])


def _make_sc_kernel(interpret=False):
    return pl.kernel(
        _sc_body,
        out_type=jax.ShapeDtypeStruct((_NW, _L), jnp.float32),
        mesh=plsc.VectorSubcoreMesh(
            core_axis_name="c", subcore_axis_name="s",
            num_cores=_NC, num_subcores=_NS),
        scratch_types=[
            pltpu.VMEM((_N,), jnp.int32),
            pltpu.VMEM((_RPW, _N), jnp.float32),
            pltpu.VMEM((_N + _L, ), jnp.float32),
            pltpu.VMEM((_L,), jnp.float32),
            pltpu.SemaphoreType.DMA,
        ],
        compiler_params=pltpu.CompilerParams(needs_layout_passes=False),
        interpret=interpret,
    )


def _sum_body(x_ref, o_ref):
    o_ref[...] = jnp.full((1, 1), jnp.sum(x_ref[...]), jnp.float32)


@jax.jit
def kernel(inputs, targets):
    inputs_t = inputs.T
    partial = _make_sc_kernel()(inputs_t, targets)
    loss = pl.pallas_call(
        _sum_body,
        out_shape=jax.ShapeDtypeStruct((1, 1), jnp.float32),
    )(partial)
    return loss[0, 0]


# speedup vs baseline: 1.5973x; 1.5973x over previous
import jax
import jax.numpy as jnp
from jax import lax
from jax.experimental import pallas as pl
from jax.experimental.pallas import tpu as pltpu
from jax.experimental.pallas import tpu_sc as plsc

_MARGIN = 0.3
_K = 8
_N = 512
_L = 16
_NC = 2
_NS = 16
_NW = _NC * _NS
_RPW = _N // _NW
_NCHUNK = _N // _L


def _sc_body(tbl_ref, tgt_ref, out_ref, t_v, rows_v, loss_v, sem):
    cid = lax.axis_index("c")
    sid = lax.axis_index("s")
    wid = sid * _NC + cid
    base = wid * _RPW

    pltpu.sync_copy(tgt_ref, t_v)
    idxv = t_v[pl.ds(base, _L)]
    pltpu.async_copy(tbl_ref.at[idxv], rows_v, sem).wait()

    iota = lax.iota(jnp.int32, _L)
    lane_lt8 = iota < _K

    pos_inf = jnp.full((_L,), jnp.inf, jnp.float32)
    neg_inf = jnp.full((_L,), -jnp.inf, jnp.float32)

    def _insert(rs, x):
        out = []
        for r in rs:
            out.append(jnp.minimum(r, x))
            x = jnp.maximum(r, x)
        return out

    def _top16(rs):
        b = jnp.sort(rs[0])
        for r in rs[1:]:
            b = jnp.sort(jnp.minimum(b, lax.rev(jnp.sort(r), (0,))))
        return b

    def row_body(l, acc):
        lane_l = iota == l
        ti = jnp.sum(jnp.where(lane_l, idxv, 0))

        def chunk(c, carry):
            mx, mn = carry[0], carry[1]
            r1s, r2s = list(carry[2:10]), list(carry[10:18])
            v = rows_v[l, pl.ds(c * _L, _L)]
            tc_ = t_v[pl.ds(c * _L, _L)]
            m = tc_ == ti
            mx = jnp.maximum(mx, v)
            mn = jnp.minimum(mn, v)
            x1 = jnp.where(m, v, jnp.inf)
            x2 = jnp.where(m, jnp.inf, -v)
            r1s = _insert(r1s, x1)
            r2s = _insert(r2s, x2)
            return (mx, mn, *r1s, *r2s)

        res = lax.fori_loop(
            0, _NCHUNK, chunk, (neg_inf, pos_inf) + (pos_inf,) * 16)
        mx, mn = res[0], res[1]
        b1 = _top16(list(res[2:10]))
        b2 = _top16(list(res[10:18]))
        max1 = jnp.max(mx)
        min2 = jnp.min(mn)
        tmp1 = jnp.minimum(b1, max1)
        tmp2 = jnp.maximum(-b2, min2)

        g_vec = rows_v[l, pl.ds(base, _L)]
        g = jnp.sum(jnp.where(lane_l, g_vec, 0.0))
        ap = jnp.abs(g - tmp1)
        an = g - tmp2

        for bi in range(_K):
            t = jnp.maximum(ap - an[bi] + _MARGIN, 0.0)
            acc = acc + jnp.where(lane_lt8, t, 0.0)
        return acc

    acc = lax.fori_loop(0, _RPW, row_body, jnp.zeros((_L,), jnp.float32))
    loss_v[...] = acc * (1.0 / (_N * _K * _K))
    pltpu.sync_copy(loss_v, out_ref.at[wid])


def _make_sc_kernel(interpret=False):
    return pl.kernel(
        _sc_body,
        out_type=jax.ShapeDtypeStruct((_NW, _L), jnp.float32),
        mesh=plsc.VectorSubcoreMesh(
            core_axis_name="c", subcore_axis_name="s",
            num_cores=_NC, num_subcores=_NS),
        scratch_types=[
            pltpu.VMEM((_N,), jnp.int32),
            pltpu.VMEM((_RPW, _N), jnp.float32),
            pltpu.VMEM((_L,), jnp.float32),
            pltpu.SemaphoreType.DMA,
        ],
        compiler_params=pltpu.CompilerParams(needs_layout_passes=False),
        interpret=interpret,
    )


def _sum_body(x_ref, o_ref):
    o_ref[...] = jnp.full((1, 1), jnp.sum(x_ref[...]), jnp.float32)


@jax.jit
def kernel(inputs, targets):
    inputs_t = inputs.T
    partial = _make_sc_kernel()(inputs_t, targets)
    loss = pl.pallas_call(
        _sum_body,
        out_shape=jax.ShapeDtypeStruct((1, 1), jnp.float32),
    )(partial)
    return loss[0, 0]


# speedup vs baseline: 1.6949x; 1.0611x over previous
import jax
import jax.numpy as jnp
from jax import lax
from jax.experimental import pallas as pl
from jax.experimental.pallas import tpu as pltpu
from jax.experimental.pallas import tpu_sc as plsc

_MARGIN = 0.3
_K = 8
_N = 512
_L = 16
_NC = 2
_NS = 16
_NW = _NC * _NS
_RPW = _N // _NW
_NCHUNK = _N // _L


def _sc_body(tbl_ref, tgt_ref, out_ref, t_v, rows_v, loss_v, sem):
    cid = lax.axis_index("c")
    sid = lax.axis_index("s")
    wid = sid * _NC + cid
    base = wid * _RPW

    pltpu.sync_copy(tgt_ref, t_v)
    idxv = t_v[pl.ds(base, _L)]
    pltpu.async_copy(tbl_ref.at[idxv], rows_v, sem).wait()

    iota = lax.iota(jnp.int32, _L)
    lane_lt8 = iota < _K

    pos_inf = jnp.full((_L,), jnp.inf, jnp.float32)
    neg_inf = jnp.full((_L,), -jnp.inf, jnp.float32)

    def _insert(rs, x):
        out = []
        for r in rs:
            out.append(jnp.minimum(r, x))
            x = jnp.maximum(r, x)
        return out

    def _m2(a, b):
        return jnp.sort(jnp.minimum(a, lax.rev(b, (0,))))

    def _top16(rs):
        s = [jnp.sort(r) for r in rs]
        while len(s) > 1:
            s = [_m2(s[i], s[i + 1]) for i in range(0, len(s), 2)]
        return s[0]

    def row_body(l, acc):
        lane_l = iota == l
        ti = jnp.sum(jnp.where(lane_l, idxv, 0))

        def chunk(c, carry):
            mx, mn, cl = carry[0], carry[1], carry[2]
            r1s, r2s = list(carry[3:5]), list(carry[5:13])
            v = rows_v[l, pl.ds(c * _L, _L)]
            tc_ = t_v[pl.ds(c * _L, _L)]
            m = tc_ == ti
            mx = jnp.maximum(mx, v)
            mn = jnp.minimum(mn, v)
            cl = cl + jnp.where(m, 1, 0)
            x1 = jnp.where(m, v, jnp.inf)
            x2 = jnp.where(m, jnp.inf, -v)
            r1s = _insert(r1s, x1)
            r2s = _insert(r2s, x2)
            return (mx, mn, cl, *r1s, *r2s)

        res = lax.fori_loop(
            0, _NCHUNK, chunk,
            (neg_inf, pos_inf, jnp.zeros((_L,), jnp.int32)) + (pos_inf,) * 10)
        mx, mn, cl = res[0], res[1], res[2]

        def b1_fast(_):
            return _top16(list(res[3:5]))

        def b1_exact(_):
            def mrg(c, b):
                v = rows_v[l, pl.ds(c * _L, _L)]
                tc_ = t_v[pl.ds(c * _L, _L)]
                ch = jnp.where(tc_ == ti, v, jnp.inf)
                return _m2(b, jnp.sort(ch))
            return lax.fori_loop(0, _NCHUNK, mrg, pos_inf)

        b1 = lax.cond(jnp.max(cl) > 2, b1_exact, b1_fast, 0)
        b2 = _top16(list(res[5:13]))
        max1 = jnp.max(mx)
        min2 = jnp.min(mn)
        tmp1 = jnp.minimum(b1, max1)
        tmp2 = jnp.maximum(-b2, min2)

        g_vec = rows_v[l, pl.ds(base, _L)]
        g = jnp.sum(jnp.where(lane_l, g_vec, 0.0))
        ap = jnp.abs(g - tmp1)
        an = g - tmp2

        for bi in range(_K):
            t = jnp.maximum(ap - an[bi] + _MARGIN, 0.0)
            acc = acc + jnp.where(lane_lt8, t, 0.0)
        return acc

    acc = lax.fori_loop(0, _RPW, row_body, jnp.zeros((_L,), jnp.float32))
    loss_v[...] = acc * (1.0 / (_N * _K * _K))
    pltpu.sync_copy(loss_v, out_ref.at[wid])


def _make_sc_kernel(interpret=False):
    return pl.kernel(
        _sc_body,
        out_type=jax.ShapeDtypeStruct((_NW, _L), jnp.float32),
        mesh=plsc.VectorSubcoreMesh(
            core_axis_name="c", subcore_axis_name="s",
            num_cores=_NC, num_subcores=_NS),
        scratch_types=[
            pltpu.VMEM((_N,), jnp.int32),
            pltpu.VMEM((_RPW, _N), jnp.float32),
            pltpu.VMEM((_L,), jnp.float32),
            pltpu.SemaphoreType.DMA,
        ],
        compiler_params=pltpu.CompilerParams(needs_layout_passes=False),
        interpret=interpret,
    )


def _sum_body(x_ref, o_ref):
    o_ref[...] = jnp.full((1, 1), jnp.sum(x_ref[...]), jnp.float32)


@jax.jit
def kernel(inputs, targets):
    inputs_t = inputs.T
    partial = _make_sc_kernel()(inputs_t, targets)
    loss = pl.pallas_call(
        _sum_body,
        out_shape=jax.ShapeDtypeStruct((1, 1), jnp.float32),
    )(partial)
    return loss[0, 0]
